# speedup vs baseline: 9.0973x; 9.0973x over previous
import jax
import jax.numpy as jnp
from jax.experimental import pallas as pl
from jax.experimental.pallas import tpu as pltpu
from jax.experimental.pallas import tpu_sc as plsc

_WINDOW = 256


def kernel(x, weight):
    batch, pos = x.shape
    vocab, embed = weight.shape
    n = batch * pos
    idx = x.reshape(1, n).astype(jnp.int32)

    mesh = plsc.VectorSubcoreMesh(core_axis_name="core", subcore_axis_name="subcore")

    @pl.kernel(
        out_type=jax.ShapeDtypeStruct((n, embed), weight.dtype),
        mesh=mesh,
    )
    def gather_kernel(w_hbm, i_hbm, o_hbm):
        def body(i_vmem, o_vmem):
            pltpu.sync_copy(w_hbm.at[i_vmem.at[0]], o_vmem)

        pltpu.emit_pipeline(
            body,
            grid=(n // _WINDOW,),
            in_specs=[pl.BlockSpec((1, _WINDOW), index_map=lambda i: (0, i))],
            out_specs=[pl.BlockSpec((_WINDOW, embed), index_map=lambda i: (i, 0))],
            core_axis_name=("core", "subcore"),
            dimension_semantics=(pltpu.PARALLEL,),
        )(i_hbm, o_hbm)

    out = gather_kernel(weight, idx)
    return out.reshape(batch, pos, embed)


# speedup vs baseline: 9.1953x; 1.0108x over previous
import functools

import jax
import jax.numpy as jnp
from jax import lax
from jax.experimental import pallas as pl
from jax.experimental.pallas import tpu as pltpu
from jax.experimental.pallas import tpu_sc as plsc

_CHUNK = 400
_NW = 32


def kernel(x, weight):
    batch, pos = x.shape
    vocab, embed = weight.shape
    n = batch * pos
    b_per_w = n // _NW
    num_chunks = b_per_w // _CHUNK
    idx = x.reshape(n).astype(jnp.int32)

    mesh = plsc.VectorSubcoreMesh(core_axis_name="c", subcore_axis_name="s")

    @functools.partial(
        pl.kernel,
        out_type=jax.ShapeDtypeStruct((n, embed), weight.dtype),
        mesh=mesh,
        scratch_types=[
            pltpu.VMEM((_CHUNK,), jnp.int32),
            pltpu.VMEM((_CHUNK,), jnp.int32),
            pltpu.VMEM((_CHUNK, embed), jnp.float32),
            pltpu.VMEM((_CHUNK, embed), jnp.float32),
            pltpu.SemaphoreType.DMA,
            pltpu.SemaphoreType.DMA,
            pltpu.SemaphoreType.DMA,
            pltpu.SemaphoreType.DMA,
        ],
    )
    def gather_kernel(w_hbm, i_hbm, o_hbm, idx0, idx1, rows0, rows1,
                      gsem0, gsem1, ssem0, ssem1):
        wid = lax.axis_index("s") * 2 + lax.axis_index("c")
        base = wid * b_per_w

        def idx_load(buf, c):
            pltpu.sync_copy(i_hbm.at[pl.ds(base + c * _CHUNK, _CHUNK)], buf)

        def gather_start(ibuf, rbuf, sem):
            pltpu.async_copy(w_hbm.at[ibuf], rbuf, sem)

        def gather_wait(ibuf, rbuf, sem):
            pltpu.make_async_copy(w_hbm.at[ibuf], rbuf, sem).wait()

        def store_start(rbuf, c, sem):
            pltpu.async_copy(rbuf, o_hbm.at[pl.ds(base + c * _CHUNK, _CHUNK)], sem)

        def store_wait(rbuf, sem):
            pltpu.make_async_copy(rbuf, o_hbm.at[pl.ds(base, _CHUNK)], sem).wait()

        idx_load(idx0, 0)
        gather_start(idx0, rows0, gsem0)
        idx_load(idx1, 1)

        @pl.loop(0, num_chunks // 2)
        def _(p):
            a = 2 * p

            gather_wait(idx0, rows0, gsem0)

            @pl.when(p > 0)
            def _():
                store_wait(rows1, ssem1)

            gather_start(idx1, rows1, gsem1)
            store_start(rows0, a, ssem0)

            @pl.when(a + 2 < num_chunks)
            def _():
                idx_load(idx0, a + 2)

            gather_wait(idx1, rows1, gsem1)
            store_wait(rows0, ssem0)

            @pl.when(a + 2 < num_chunks)
            def _():
                gather_start(idx0, rows0, gsem0)

            store_start(rows1, a + 1, ssem1)

            @pl.when(a + 3 < num_chunks)
            def _():
                idx_load(idx1, a + 3)

        store_wait(rows1, ssem1)

    out = gather_kernel(weight, idx)
    return out.reshape(batch, pos, embed)


# speedup vs baseline: 9.2405x; 1.0049x over previous
import functools

import jax
import jax.numpy as jnp
from jax import lax
from jax.experimental import pallas as pl
from jax.experimental.pallas import tpu as pltpu
from jax.experimental.pallas import tpu_sc as plsc

_CHUNK = 200
_NB = 4
_NW = 32


def kernel(x, weight):
    batch, pos = x.shape
    vocab, embed = weight.shape
    n = batch * pos
    b_per_w = n // _NW
    num_chunks = b_per_w // _CHUNK
    idx = x.reshape(n).astype(jnp.int32)

    mesh = plsc.VectorSubcoreMesh(core_axis_name="c", subcore_axis_name="s")

    @functools.partial(
        pl.kernel,
        out_type=jax.ShapeDtypeStruct((n, embed), weight.dtype),
        mesh=mesh,
        scratch_types=(
            [pltpu.VMEM((_CHUNK,), jnp.int32) for _ in range(_NB)]
            + [pltpu.VMEM((_CHUNK, embed), jnp.float32) for _ in range(_NB)]
            + [pltpu.SemaphoreType.DMA for _ in range(2 * _NB)]
        ),
    )
    def gather_kernel(w_hbm, i_hbm, o_hbm, *scratch):
        idxb = scratch[:_NB]
        rows = scratch[_NB:2 * _NB]
        gsem = scratch[2 * _NB:3 * _NB]
        ssem = scratch[3 * _NB:4 * _NB]

        wid = lax.axis_index("s") * 2 + lax.axis_index("c")
        base = wid * b_per_w

        def idx_load(j, c):
            pltpu.sync_copy(i_hbm.at[pl.ds(base + c * _CHUNK, _CHUNK)], idxb[j])

        def gather_start(j):
            pltpu.async_copy(w_hbm.at[idxb[j]], rows[j], gsem[j])

        def gather_wait(j):
            pltpu.make_async_copy(w_hbm.at[idxb[j]], rows[j], gsem[j]).wait()

        def store_start(j, c):
            pltpu.async_copy(rows[j], o_hbm.at[pl.ds(base + c * _CHUNK, _CHUNK)],
                             ssem[j])

        def store_wait(j):
            pltpu.make_async_copy(rows[j], o_hbm.at[pl.ds(base, _CHUNK)],
                                  ssem[j]).wait()

        idx_load(0, 0)
        gather_start(0)
        idx_load(1, 1)
        gather_start(1)

        last_full = num_chunks // _NB - 1

        @pl.loop(0, num_chunks // _NB)
        def _(p):
            cc = _NB * p
            for r in range(_NB):
                c = cc + r
                nxt = (r + 2) % _NB

                gather_wait(r)
                store_start(r, c)

                if r < 2:
                    @pl.when(p > 0)
                    def _():
                        store_wait(nxt)

                    idx_load(nxt, c + 2)
                    gather_start(nxt)
                else:
                    store_wait(nxt)

                    @pl.when(p < last_full)
                    def _():
                        idx_load(nxt, c + 2)
                        gather_start(nxt)

        store_wait(2)
        store_wait(3)

    out = gather_kernel(weight, idx)
    return out.reshape(batch, pos, embed)


# speedup vs baseline: 9.2480x; 1.0008x over previous
import functools

import jax
import jax.numpy as jnp
from jax import lax
from jax.experimental import pallas as pl
from jax.experimental.pallas import tpu as pltpu
from jax.experimental.pallas import tpu_sc as plsc

_CHUNK = 160
_NB = 4
_NW = 32


def kernel(x, weight):
    batch, pos = x.shape
    vocab, embed = weight.shape
    n = batch * pos
    b_per_w = n // _NW
    num_chunks = b_per_w // _CHUNK
    idx = x.reshape(n).astype(jnp.int32)

    mesh = plsc.VectorSubcoreMesh(core_axis_name="c", subcore_axis_name="s")

    @functools.partial(
        pl.kernel,
        out_type=jax.ShapeDtypeStruct((n, embed), weight.dtype),
        mesh=mesh,
        scratch_types=(
            [pltpu.VMEM((_CHUNK,), jnp.int32) for _ in range(_NB)]
            + [pltpu.VMEM((_CHUNK, embed), jnp.float32) for _ in range(_NB)]
            + [pltpu.SemaphoreType.DMA for _ in range(2 * _NB)]
        ),
    )
    def gather_kernel(w_hbm, i_hbm, o_hbm, *scratch):
        idxb = scratch[:_NB]
        rows = scratch[_NB:2 * _NB]
        gsem = scratch[2 * _NB:3 * _NB]
        ssem = scratch[3 * _NB:4 * _NB]

        wid = lax.axis_index("s") * 2 + lax.axis_index("c")
        base = wid * b_per_w

        def idx_load(j, c):
            pltpu.sync_copy(i_hbm.at[pl.ds(base + c * _CHUNK, _CHUNK)], idxb[j])

        def gather_start(j):
            pltpu.async_copy(w_hbm.at[idxb[j]], rows[j], gsem[j])

        def gather_wait(j):
            pltpu.make_async_copy(w_hbm.at[idxb[j]], rows[j], gsem[j]).wait()

        def store_start(j, c):
            pltpu.async_copy(rows[j], o_hbm.at[pl.ds(base + c * _CHUNK, _CHUNK)],
                             ssem[j])

        def store_wait(j):
            pltpu.make_async_copy(rows[j], o_hbm.at[pl.ds(base, _CHUNK)],
                                  ssem[j]).wait()

        idx_load(0, 0)
        gather_start(0)
        idx_load(1, 1)
        gather_start(1)

        last_full = num_chunks // _NB - 1

        @pl.loop(0, num_chunks // _NB)
        def _(p):
            cc = _NB * p
            for r in range(_NB):
                c = cc + r
                nxt = (r + 2) % _NB

                gather_wait(r)
                store_start(r, c)

                if r < 2:
                    @pl.when(p > 0)
                    def _():
                        store_wait(nxt)

                    idx_load(nxt, c + 2)
                    gather_start(nxt)
                else:
                    store_wait(nxt)

                    @pl.when(p < last_full)
                    def _():
                        idx_load(nxt, c + 2)
                        gather_start(nxt)

        store_wait(2)
        store_wait(3)

    out = gather_kernel(weight, idx)
    return out.reshape(batch, pos, embed)
